# Initial kernel scaffold; baseline (speedup 1.0000x reference)
import jax
import jax.numpy as jnp
from jax.experimental import pallas as pl


def kernel(inputs, table):
    raise NotImplementedError("write your pallas kernel here")


# speedup vs baseline: 1.5496x; 1.5496x over previous
import functools

import jax
import jax.numpy as jnp
from jax import lax
from jax.experimental import pallas as pl
from jax.experimental.pallas import tpu as pltpu
from jax.experimental.pallas import tpu_sc as plsc

EMBED_DIM = 32
NUM_CORES = 2
NUM_SUBCORES = 16
NW = NUM_CORES * NUM_SUBCORES
CHUNK = 1024


def _sc_gather(table, idx3, nch):
    n = NW * nch * CHUNK
    mesh = plsc.VectorSubcoreMesh(core_axis_name="c", subcore_axis_name="s")

    @functools.partial(
        pl.kernel,
        mesh=mesh,
        out_type=jax.ShapeDtypeStruct((n, EMBED_DIM), jnp.float32),
        scratch_types=[
            pltpu.VMEM((CHUNK,), jnp.int32),
            pltpu.VMEM((CHUNK, EMBED_DIM), jnp.float32),
            pltpu.SemaphoreType.DMA,
        ],
        compiler_params=pltpu.CompilerParams(use_tc_tiling_on_sc=False),
    )
    def k(table_hbm, idx_hbm, out_hbm, idx_v, rows_v, sem):
        wid = lax.axis_index("s") * NUM_CORES + lax.axis_index("c")
        base = wid * (nch * CHUNK)

        def body(j, carry):
            pltpu.sync_copy(idx_hbm.at[wid, j], idx_v)
            pltpu.async_copy(table_hbm.at[idx_v], rows_v, sem).wait()
            pltpu.sync_copy(rows_v, out_hbm.at[pl.ds(base + j * CHUNK, CHUNK)])
            return carry

        lax.fori_loop(0, nch, body, 0)

    return k(table, idx3)


def kernel(inputs, table):
    b, f = inputs.shape
    total = b * f
    nch = total // (NW * CHUNK)
    idx3 = inputs.astype(jnp.int32).reshape(NW, nch, CHUNK)
    out = _sc_gather(table, idx3, nch)
    return out.reshape(b, f, EMBED_DIM)


# speedup vs baseline: 1.5754x; 1.0167x over previous
import functools

import jax
import jax.numpy as jnp
from jax import lax
from jax.experimental import pallas as pl
from jax.experimental.pallas import tpu as pltpu
from jax.experimental.pallas import tpu_sc as plsc

EMBED_DIM = 32
NUM_CORES = 2
NUM_SUBCORES = 16
NW = NUM_CORES * NUM_SUBCORES
CHUNK = 1024


NBUF = 3


def _sc_gather(table, idx3, nch):
    n = NW * nch * CHUNK
    mesh = plsc.VectorSubcoreMesh(core_axis_name="c", subcore_axis_name="s")

    @functools.partial(
        pl.kernel,
        mesh=mesh,
        out_type=jax.ShapeDtypeStruct((n, EMBED_DIM), jnp.float32),
        scratch_types=[
            pltpu.VMEM((nch, CHUNK), jnp.int32),
            [pltpu.VMEM((CHUNK, EMBED_DIM), jnp.float32) for _ in range(NBUF)],
            [pltpu.SemaphoreType.DMA for _ in range(NBUF)],
            [pltpu.SemaphoreType.DMA for _ in range(NBUF)],
        ],
        compiler_params=pltpu.CompilerParams(use_tc_tiling_on_sc=False),
    )
    def k(table_hbm, idx_hbm, out_hbm, idx_v, rows, gsem, wsem):
        wid = lax.axis_index("s") * NUM_CORES + lax.axis_index("c")
        base = wid * (nch * CHUNK)
        pltpu.sync_copy(idx_hbm.at[wid], idx_v)

        gh, wh = {}, {}
        for j in range(min(NBUF, nch)):
            gh[j] = pltpu.async_copy(
                table_hbm.at[idx_v.at[j]], rows[j % NBUF], gsem[j % NBUF])
        for j in range(nch):
            b = j % NBUF
            gh[j].wait()
            wh[j] = pltpu.async_copy(
                rows[b], out_hbm.at[pl.ds(base + j * CHUNK, CHUNK)], wsem[b])
            wh[j].wait()
            if j + NBUF < nch:
                gh[j + NBUF] = pltpu.async_copy(
                    table_hbm.at[idx_v.at[j + NBUF]], rows[b], gsem[b])

    return k(table, idx3)


def kernel(inputs, table):
    b, f = inputs.shape
    total = b * f
    nch = total // (NW * CHUNK)
    idx3 = inputs.astype(jnp.int32).reshape(NW, nch, CHUNK)
    out = _sc_gather(table, idx3, nch)
    return out.reshape(b, f, EMBED_DIM)


# speedup vs baseline: 1.5763x; 1.0005x over previous
import functools

import jax
import jax.numpy as jnp
from jax import lax
from jax.experimental import pallas as pl
from jax.experimental.pallas import tpu as pltpu
from jax.experimental.pallas import tpu_sc as plsc

EMBED_DIM = 32
NUM_CORES = 2
NUM_SUBCORES = 16
NW = NUM_CORES * NUM_SUBCORES
CHUNK = 1024


NBUF = 3


def _sc_gather(table, idx3, nch):
    n = NW * nch * CHUNK
    mesh = plsc.VectorSubcoreMesh(core_axis_name="c", subcore_axis_name="s")

    @functools.partial(
        pl.kernel,
        mesh=mesh,
        out_type=jax.ShapeDtypeStruct((n, EMBED_DIM), jnp.float32),
        scratch_types=[
            pltpu.VMEM((nch, CHUNK), jnp.int32),
            [pltpu.VMEM((CHUNK, EMBED_DIM), jnp.float32) for _ in range(NBUF)],
            [pltpu.SemaphoreType.DMA for _ in range(NBUF)],
            [pltpu.SemaphoreType.DMA for _ in range(NBUF)],
        ],
        compiler_params=pltpu.CompilerParams(use_tc_tiling_on_sc=False),
    )
    def k(table_hbm, idx_hbm, out_hbm, idx_v, rows, gsem, wsem):
        wid = lax.axis_index("s") * NUM_CORES + lax.axis_index("c")
        base = wid * (nch * CHUNK)
        pltpu.sync_copy(idx_hbm.at[wid], idx_v)

        gh, wh = {}, {}
        for j in range(min(NBUF, nch)):
            gh[j] = pltpu.async_copy(
                table_hbm.at[idx_v.at[j]], rows[j % NBUF], gsem[j % NBUF])
        for j in range(nch):
            b = j % NBUF
            gh[j].wait()
            wh[j] = pltpu.async_copy(
                rows[b], out_hbm.at[pl.ds(base + j * CHUNK, CHUNK)], wsem[b])
            wh[j].wait()
            if j + NBUF < nch:
                gh[j + NBUF] = pltpu.async_copy(
                    table_hbm.at[idx_v.at[j + NBUF]], rows[b], gsem[b])

    return k(table, idx3)


def kernel(inputs, table):
    b, f = inputs.shape
    total = b * f
    nch = total // (NW * CHUNK)
    idx3 = inputs.astype(jnp.int32).reshape(NW, nch, CHUNK)
    vocab = table.shape[0]
    tab_rm = jax.lax.optimization_barrier(
        table.reshape(vocab // 4, 4 * EMBED_DIM))
    out = _sc_gather(tab_rm.reshape(vocab, EMBED_DIM), idx3, nch)
    return out.reshape(b, f, EMBED_DIM)


# speedup vs baseline: 1.5997x; 1.0149x over previous
import functools

import jax
import jax.numpy as jnp
from jax import lax
from jax.experimental import pallas as pl
from jax.experimental.pallas import tpu as pltpu
from jax.experimental.pallas import tpu_sc as plsc

EMBED_DIM = 32
NUM_CORES = 2
NUM_SUBCORES = 16
NW = NUM_CORES * NUM_SUBCORES
CHUNK = 1024


NBUF = 3


def _sc_gather(table, idx3, nch, batch, fields):
    mesh = plsc.VectorSubcoreMesh(core_axis_name="c", subcore_axis_name="s")

    @functools.partial(
        pl.kernel,
        mesh=mesh,
        out_type=jax.ShapeDtypeStruct(
            (batch * fields, EMBED_DIM), jnp.float32),
        scratch_types=[
            pltpu.VMEM((nch, CHUNK), jnp.int32),
            [pltpu.VMEM((CHUNK, EMBED_DIM), jnp.float32) for _ in range(NBUF)],
            [pltpu.SemaphoreType.DMA for _ in range(NBUF)],
            [pltpu.SemaphoreType.DMA for _ in range(NBUF)],
        ],
        compiler_params=pltpu.CompilerParams(use_tc_tiling_on_sc=False),
    )
    def k(table_hbm, idx_hbm, out_hbm, idx_v, rows, gsem, wsem):
        wid = lax.axis_index("s") * NUM_CORES + lax.axis_index("c")
        base = wid * (nch * CHUNK)
        pltpu.sync_copy(idx_hbm.at[wid], idx_v)

        gh, wh = {}, {}
        for j in range(min(NBUF, nch)):
            gh[j] = pltpu.async_copy(
                table_hbm.at[idx_v.at[j]], rows[j % NBUF], gsem[j % NBUF])
        for j in range(nch):
            b = j % NBUF
            gh[j].wait()
            wh[j] = pltpu.async_copy(
                rows[b], out_hbm.at[pl.ds(base + j * CHUNK, CHUNK)], wsem[b])
            wh[j].wait()
            if j + NBUF < nch:
                gh[j + NBUF] = pltpu.async_copy(
                    table_hbm.at[idx_v.at[j + NBUF]], rows[b], gsem[b])

    return k(table, idx3)


def kernel(inputs, table):
    b, f = inputs.shape
    total = b * f
    nch = total // (NW * CHUNK)
    idx3 = inputs.astype(jnp.int32).reshape(NW, nch, CHUNK) * 4
    vocab = table.shape[0]
    tab_pad = jnp.pad(table, ((0, 0), (0, 128 - EMBED_DIM)))
    tabv = tab_pad.reshape(vocab * 4, EMBED_DIM)
    out = _sc_gather(tabv, idx3, nch, b, f)
    return out.reshape(b, f, EMBED_DIM)


# speedup vs baseline: 1.9947x; 1.2469x over previous
import functools

import jax
import jax.numpy as jnp
from jax import lax
from jax.experimental import pallas as pl
from jax.experimental.pallas import tpu as pltpu
from jax.experimental.pallas import tpu_sc as plsc

EMBED_DIM = 32
NUM_CORES = 2
NUM_SUBCORES = 16
NW = NUM_CORES * NUM_SUBCORES
NBUF = 3


def _sc_gather(table3, idxp, batch, fields, fpad, cpad):
    bpw = batch // NW
    mesh = plsc.VectorSubcoreMesh(core_axis_name="c", subcore_axis_name="s")

    @functools.partial(
        pl.kernel,
        mesh=mesh,
        out_type=jax.ShapeDtypeStruct((batch, fpad, cpad), jnp.float32),
        scratch_types=[
            pltpu.VMEM((fields, bpw), jnp.int32),
            [pltpu.VMEM((bpw, EMBED_DIM), jnp.float32) for _ in range(NBUF)],
            [pltpu.SemaphoreType.DMA for _ in range(NBUF)],
            [pltpu.SemaphoreType.DMA for _ in range(NBUF)],
        ],
        compiler_params=pltpu.CompilerParams(use_tc_tiling_on_sc=False),
    )
    def k(table_hbm, idx_hbm, out_hbm, idx_v, rows, gsem, wsem):
        wid = lax.axis_index("s") * NUM_CORES + lax.axis_index("c")
        b0 = wid * bpw
        pltpu.sync_copy(idx_hbm.at[wid], idx_v)

        gh, wh = {}, {}
        for f in range(min(NBUF, fields)):
            gh[f] = pltpu.async_copy(
                table_hbm.at[idx_v.at[f]], rows[f % NBUF], gsem[f % NBUF])
        for f in range(fields):
            b = f % NBUF
            gh[f].wait()
            wh[f] = pltpu.async_copy(
                rows[b],
                out_hbm.at[pl.ds(b0, bpw), f, pl.ds(0, EMBED_DIM)],
                wsem[b])
            wh[f].wait()
            if f + NBUF < fields:
                gh[f + NBUF] = pltpu.async_copy(
                    table_hbm.at[idx_v.at[f + NBUF]], rows[b], gsem[b])

    return k(table3, idxp)


def kernel(inputs, table):
    batch, fields = inputs.shape
    vocab = table.shape[0]
    fpad = 32
    cpad = 128
    idxp = (inputs.astype(jnp.int32).T
            .reshape(fields, NW, batch // NW)
            .transpose(1, 0, 2))
    tab_rm = jax.lax.optimization_barrier(
        table.reshape(vocab // 4, 4 * EMBED_DIM))
    out5 = _sc_gather(
        tab_rm.reshape(vocab, EMBED_DIM), idxp, batch, fields, fpad, cpad)
    return out5[:, :fields, :EMBED_DIM]


# speedup vs baseline: 1.9969x; 1.0011x over previous
import functools

import jax
import jax.numpy as jnp
from jax import lax
from jax.experimental import pallas as pl
from jax.experimental.pallas import tpu as pltpu
from jax.experimental.pallas import tpu_sc as plsc

EMBED_DIM = 32
NUM_CORES = 2
NUM_SUBCORES = 16
NW = NUM_CORES * NUM_SUBCORES
NBUF = 5


def _sc_gather(table3, idxp, batch, fields, fpad, cpad):
    bpw = batch // NW
    mesh = plsc.VectorSubcoreMesh(core_axis_name="c", subcore_axis_name="s")

    @functools.partial(
        pl.kernel,
        mesh=mesh,
        out_type=jax.ShapeDtypeStruct((batch, fpad, cpad), jnp.float32),
        scratch_types=[
            pltpu.VMEM((fields, bpw), jnp.int32),
            [pltpu.VMEM((bpw, EMBED_DIM), jnp.float32) for _ in range(NBUF)],
            [pltpu.SemaphoreType.DMA for _ in range(NBUF)],
            [pltpu.SemaphoreType.DMA for _ in range(NBUF)],
        ],
        compiler_params=pltpu.CompilerParams(use_tc_tiling_on_sc=False),
    )
    def k(table_hbm, idx_hbm, out_hbm, idx_v, rows, gsem, wsem):
        wid = lax.axis_index("s") * NUM_CORES + lax.axis_index("c")
        b0 = wid * bpw
        pltpu.sync_copy(idx_hbm.at[wid], idx_v)

        gh, wh = {}, {}
        for f in range(min(NBUF, fields)):
            gh[f] = pltpu.async_copy(
                table_hbm.at[idx_v.at[f]], rows[f % NBUF], gsem[f % NBUF])
        for f in range(fields):
            b = f % NBUF
            gh[f].wait()
            wh[f] = pltpu.async_copy(
                rows[b],
                out_hbm.at[pl.ds(b0, bpw), f, pl.ds(0, EMBED_DIM)],
                wsem[b])
            wh[f].wait()
            if f + NBUF < fields:
                gh[f + NBUF] = pltpu.async_copy(
                    table_hbm.at[idx_v.at[f + NBUF]], rows[b], gsem[b])

    return k(table3, idxp)


def kernel(inputs, table):
    batch, fields = inputs.shape
    vocab = table.shape[0]
    fpad = 32
    cpad = 128
    idxp = (inputs.astype(jnp.int32).T
            .reshape(fields, NW, batch // NW)
            .transpose(1, 0, 2))
    tab_rm = jax.lax.optimization_barrier(
        table.reshape(vocab // 4, 4 * EMBED_DIM))
    out5 = _sc_gather(
        tab_rm.reshape(vocab, EMBED_DIM), idxp, batch, fields, fpad, cpad)
    return out5[:, :fields, :EMBED_DIM]
